# baseline (speedup 1.0000x reference)
import jax
import jax.numpy as jnp
from jax.experimental import pallas as pl
from jax.experimental.pallas import tpu as pltpu

_P = 16


def _vad_gather_kernel(idx_ref, tab_hbm, eps_ref,
                       mu_ref, lv_ref, std_ref, lat_ref, aug_ref,
                       rows, sem):
    tb = mu_ref.shape[0]
    nrows = tab_hbm.shape[0]
    base = pl.program_id(0) * tb
    for i in range(tb):
        r = jnp.clip(idx_ref[base + i], 0, nrows - 1)
        pltpu.make_async_copy(tab_hbm.at[r], rows.at[i], sem).start()
    pltpu.make_async_copy(tab_hbm.at[pl.ds(0, tb)],
                          rows.at[pl.ds(0, tb)], sem).wait()

    mu = rows[:, 0, :]
    logvar = rows[:, 1, :]
    std = jnp.exp(0.5 * logvar)
    eps = eps_ref[...]
    mu_ref[...] = mu
    lv_ref[...] = logvar
    std_ref[...] = std
    lat_ref[...] = mu + eps[:, _P, :] * std
    aug_ref[...] = mu[:, None, :] + eps[:, :_P, :] * std[:, None, :]


def kernel(idx, tab_fused, eps_seed):
    b = int(idx.shape[0])
    n_pad, two_dim = tab_fused.shape
    dim = two_dim // 2
    tab3 = tab_fused.reshape(n_pad, 2, dim)

    eps_all = jax.random.normal(jax.random.key(eps_seed),
                                (b, _P + 1, dim), dtype=jnp.float32)

    nsteps = 2 if b % 2 == 0 else 1
    tb = b // nsteps

    grid_spec = pltpu.PrefetchScalarGridSpec(
        num_scalar_prefetch=1,
        grid=(nsteps,),
        in_specs=[
            pl.BlockSpec(memory_space=pl.ANY),
            pl.BlockSpec((tb, _P + 1, dim), lambda g, sref: (g, 0, 0)),
        ],
        out_specs=[
            pl.BlockSpec((tb, dim), lambda g, sref: (g, 0)),
            pl.BlockSpec((tb, dim), lambda g, sref: (g, 0)),
            pl.BlockSpec((tb, dim), lambda g, sref: (g, 0)),
            pl.BlockSpec((tb, dim), lambda g, sref: (g, 0)),
            pl.BlockSpec((tb, _P, dim), lambda g, sref: (g, 0, 0)),
        ],
        scratch_shapes=[
            pltpu.VMEM((tb, 2, dim), jnp.float32),
            pltpu.SemaphoreType.DMA,
        ],
    )
    out_shape = (tuple(jax.ShapeDtypeStruct((b, dim), jnp.float32)
                       for _ in range(4))
                 + (jax.ShapeDtypeStruct((b, _P, dim), jnp.float32),))
    mu, logvar, std, latent, latent_aug = pl.pallas_call(
        _vad_gather_kernel,
        grid_spec=grid_spec,
        out_shape=out_shape,
        compiler_params=pltpu.CompilerParams(
            dimension_semantics=("parallel",)),
    )(idx.astype(jnp.int32), tab3, eps_all)

    return {'latent_code': latent,
            'latent_code_augment': latent_aug,
            'mu': mu, 'logvar': logvar, 'std': std}


# speedup vs baseline: 1.5724x; 1.5724x over previous
import jax
import jax.numpy as jnp
from jax.experimental import pallas as pl
from jax.experimental.pallas import tpu as pltpu

_P = 16


def _vad_gather_kernel(idx_ref, tab_hbm, idxv_ref, eps_ref,
                       mu_ref, lv_ref, std_ref, lat_ref, aug_ref,
                       chunks, sem):
    tb = mu_ref.shape[0]
    dim = mu_ref.shape[1]
    nrows = tab_hbm.shape[0]
    base = pl.program_id(0) * tb
    for i in range(tb):
        r = jnp.clip(idx_ref[base + i], 0, nrows - 1)
        c = pl.multiple_of((r >> 3) << 3, 8)
        pltpu.make_async_copy(tab_hbm.at[pl.ds(c, 8), :],
                              chunks.at[pl.ds(i * 8, 8), :], sem).start()
    pltpu.make_async_copy(tab_hbm.at[pl.ds(0, 8 * tb), :],
                          chunks.at[pl.ds(0, 8 * tb), :], sem).wait()

    blk = chunks[...].reshape(tb, 8, 2 * dim)
    rem = (idxv_ref[...] & 7).reshape(tb, 1, 1)
    sub = jax.lax.broadcasted_iota(jnp.int32, (tb, 8, 2 * dim), 1)
    picked = jnp.sum(blk * (sub == rem).astype(jnp.float32), axis=1)

    mu = picked[:, :dim]
    logvar = picked[:, dim:]
    std = jnp.exp(0.5 * logvar)
    eps = eps_ref[...]
    mu_ref[...] = mu
    lv_ref[...] = logvar
    std_ref[...] = std
    lat_ref[...] = mu + eps[:, _P, :] * std
    aug_ref[...] = mu[:, None, :] + eps[:, :_P, :] * std[:, None, :]


def kernel(idx, tab_fused, eps_seed):
    b = int(idx.shape[0])
    n_pad, two_dim = tab_fused.shape
    dim = two_dim // 2

    eps_all = jax.random.normal(jax.random.key(eps_seed),
                                (b, _P + 1, dim), dtype=jnp.float32)

    idx32 = idx.astype(jnp.int32)
    idx_col = idx32.reshape(b, 1)

    nsteps = 2 if b % 2 == 0 else 1
    tb = b // nsteps

    grid_spec = pltpu.PrefetchScalarGridSpec(
        num_scalar_prefetch=1,
        grid=(nsteps,),
        in_specs=[
            pl.BlockSpec(memory_space=pl.ANY),
            pl.BlockSpec((tb, 1), lambda g, sref: (g, 0)),
            pl.BlockSpec((tb, _P + 1, dim), lambda g, sref: (g, 0, 0)),
        ],
        out_specs=[
            pl.BlockSpec((tb, dim), lambda g, sref: (g, 0)),
            pl.BlockSpec((tb, dim), lambda g, sref: (g, 0)),
            pl.BlockSpec((tb, dim), lambda g, sref: (g, 0)),
            pl.BlockSpec((tb, dim), lambda g, sref: (g, 0)),
            pl.BlockSpec((tb, _P, dim), lambda g, sref: (g, 0, 0)),
        ],
        scratch_shapes=[
            pltpu.VMEM((tb * 8, two_dim), jnp.float32),
            pltpu.SemaphoreType.DMA,
        ],
    )
    out_shape = (tuple(jax.ShapeDtypeStruct((b, dim), jnp.float32)
                       for _ in range(4))
                 + (jax.ShapeDtypeStruct((b, _P, dim), jnp.float32),))
    mu, logvar, std, latent, latent_aug = pl.pallas_call(
        _vad_gather_kernel,
        grid_spec=grid_spec,
        out_shape=out_shape,
        compiler_params=pltpu.CompilerParams(
            dimension_semantics=("parallel",)),
    )(idx32, tab_fused, idx_col, eps_all)

    return {'latent_code': latent,
            'latent_code_augment': latent_aug,
            'mu': mu, 'logvar': logvar, 'std': std}


# speedup vs baseline: 2.0640x; 1.3126x over previous
import numpy as np

import jax
import jax.numpy as jnp
from jax.experimental import pallas as pl
from jax.experimental.pallas import tpu as pltpu

_P = 16
_CB = 8

_ROT_A = (13, 15, 26, 6)
_ROT_B = (17, 29, 16, 24)

_LO = np.nextafter(np.float32(-1.0), np.float32(0.0), dtype=np.float32)
_SPAN = np.float32(np.float32(1.0) - _LO)
_SQRT2 = np.float32(np.sqrt(2.0))


def _rotl(x, r):
    return jax.lax.shift_left(x, jnp.uint32(r)) | jax.lax.shift_right_logical(
        x, jnp.uint32(32 - r))


def _threefry_bits(k0, k1, ks2, x1):
    x0 = k0
    x1 = x1 + k1
    inject = ((k1, ks2), (ks2, k0), (k0, k1), (k1, ks2), (ks2, k0))
    for i, rots in enumerate((_ROT_A, _ROT_B, _ROT_A, _ROT_B, _ROT_A)):
        for r in rots:
            x0 = x0 + x1
            x1 = _rotl(x1, r)
            x1 = x0 ^ x1
        a, c = inject[i]
        x0 = x0 + a
        x1 = x1 + (c + jnp.uint32(i + 1))
    return x0 ^ x1


def _eps_from_counts(k0, k1, ks2, lin_i32):
    bits = _threefry_bits(k0, k1, ks2, lin_i32.astype(jnp.uint32))
    fb = jax.lax.shift_right_logical(bits, jnp.uint32(9)) | jnp.uint32(
        0x3F800000)
    u01 = pltpu.bitcast(fb, jnp.float32) - jnp.float32(1.0)
    u = jnp.maximum(jnp.float32(_LO), u01 * jnp.float32(_SPAN)
                    + jnp.float32(_LO))
    return jnp.float32(_SQRT2) * jax.lax.erf_inv(u)


def _vad_kernel(idx_ref, kd_ref, tab_hbm, idxv_ref,
                mu_ref, lv_ref, std_ref, lat_ref, aug_ref,
                chunks, sem):
    tb = mu_ref.shape[0]
    dim = mu_ref.shape[1]
    nrows = tab_hbm.shape[0]
    row_elems = (_P + 1) * dim
    base_row = pl.program_id(0) * tb

    for i in range(tb):
        r = jnp.clip(idx_ref[base_row + i], 0, nrows - 1)
        c = pl.multiple_of((r >> 3) << 3, 8)
        pltpu.make_async_copy(tab_hbm.at[pl.ds(c, 8), :],
                              chunks.at[pl.ds(i * 8, 8), :], sem).start()

    k0 = kd_ref[0].astype(jnp.uint32)
    k1 = kd_ref[1].astype(jnp.uint32)
    ks2 = k0 ^ k1 ^ jnp.uint32(0x1BD11BDA)

    lin = (jax.lax.broadcasted_iota(jnp.int32, (tb, dim), 0) * row_elems
           + jax.lax.broadcasted_iota(jnp.int32, (tb, dim), 1)
           + (base_row * row_elems + _P * dim))
    eps_lat = _eps_from_counts(k0, k1, ks2, lin)

    pltpu.make_async_copy(tab_hbm.at[pl.ds(0, 8 * tb), :],
                          chunks.at[pl.ds(0, 8 * tb), :], sem).wait()
    blk = chunks[...].reshape(tb, 8, 2 * dim)
    rem = (idxv_ref[...] & 7).reshape(tb, 1, 1)
    sub = jax.lax.broadcasted_iota(jnp.int32, (tb, 8, 2 * dim), 1)
    picked = jnp.sum(blk * (sub == rem).astype(jnp.float32), axis=1)

    mu = picked[:, :dim]
    logvar = picked[:, dim:]
    std = jnp.exp(0.5 * logvar)
    mu_ref[...] = mu
    lv_ref[...] = logvar
    std_ref[...] = std
    lat_ref[...] = mu + eps_lat * std

    def aug_body(i, carry):
        b0 = pl.multiple_of(i * _CB, _CB)
        lin = (jax.lax.broadcasted_iota(jnp.int32, (_CB, _P, dim), 0)
               * row_elems
               + jax.lax.broadcasted_iota(jnp.int32, (_CB, _P, dim), 1) * dim
               + jax.lax.broadcasted_iota(jnp.int32, (_CB, _P, dim), 2)
               + (base_row + b0) * row_elems)
        eps = _eps_from_counts(k0, k1, ks2, lin)
        mu8 = mu_ref[pl.ds(b0, _CB), :]
        std8 = std_ref[pl.ds(b0, _CB), :]
        aug_ref[pl.ds(b0, _CB), :, :] = (mu8[:, None, :]
                                         + eps * std8[:, None, :])
        return carry

    jax.lax.fori_loop(0, tb // _CB, aug_body, 0)


def kernel(idx, tab_fused, eps_seed):
    b = int(idx.shape[0])
    n_pad, two_dim = tab_fused.shape
    dim = two_dim // 2

    kd = jax.lax.bitcast_convert_type(
        jax.random.key_data(jax.random.key(eps_seed)), jnp.int32)

    idx32 = idx.astype(jnp.int32)
    idx_col = idx32.reshape(b, 1)

    nsteps = 2 if b % 2 == 0 else 1
    tb = b // nsteps

    grid_spec = pltpu.PrefetchScalarGridSpec(
        num_scalar_prefetch=2,
        grid=(nsteps,),
        in_specs=[
            pl.BlockSpec(memory_space=pl.ANY),
            pl.BlockSpec((tb, 1), lambda g, *_: (g, 0)),
        ],
        out_specs=[
            pl.BlockSpec((tb, dim), lambda g, *_: (g, 0)),
            pl.BlockSpec((tb, dim), lambda g, *_: (g, 0)),
            pl.BlockSpec((tb, dim), lambda g, *_: (g, 0)),
            pl.BlockSpec((tb, dim), lambda g, *_: (g, 0)),
            pl.BlockSpec((tb, _P, dim), lambda g, *_: (g, 0, 0)),
        ],
        scratch_shapes=[
            pltpu.VMEM((tb * 8, two_dim), jnp.float32),
            pltpu.SemaphoreType.DMA,
        ],
    )
    out_shape = (tuple(jax.ShapeDtypeStruct((b, dim), jnp.float32)
                       for _ in range(4))
                 + (jax.ShapeDtypeStruct((b, _P, dim), jnp.float32),))
    mu, logvar, std, latent, latent_aug = pl.pallas_call(
        _vad_kernel,
        grid_spec=grid_spec,
        out_shape=out_shape,
        compiler_params=pltpu.CompilerParams(
            dimension_semantics=("parallel",)),
    )(idx32, kd, tab_fused, idx_col)

    return {'latent_code': latent,
            'latent_code_augment': latent_aug,
            'mu': mu, 'logvar': logvar, 'std': std}


# speedup vs baseline: 2.1421x; 1.0378x over previous
import numpy as np

import jax
import jax.numpy as jnp
from jax.experimental import pallas as pl
from jax.experimental.pallas import tpu as pltpu

_P = 16
_CB = 16

_ROT_A = (13, 15, 26, 6)
_ROT_B = (17, 29, 16, 24)

_LO = np.nextafter(np.float32(-1.0), np.float32(0.0), dtype=np.float32)
_SPAN = np.float32(np.float32(1.0) - _LO)
_SQRT2 = np.float32(np.sqrt(2.0))


def _rotl(x, r):
    return jax.lax.shift_left(x, jnp.uint32(r)) | jax.lax.shift_right_logical(
        x, jnp.uint32(32 - r))


def _threefry_bits(k0, k1, ks2, x1):
    x0 = k0
    x1 = x1 + k1
    inject = ((k1, ks2), (ks2, k0), (k0, k1), (k1, ks2), (ks2, k0))
    for i, rots in enumerate((_ROT_A, _ROT_B, _ROT_A, _ROT_B, _ROT_A)):
        for r in rots:
            x0 = x0 + x1
            x1 = _rotl(x1, r)
            x1 = x0 ^ x1
        a, c = inject[i]
        x0 = x0 + a
        x1 = x1 + (c + jnp.uint32(i + 1))
    return x0 ^ x1


def _eps_from_counts(k0, k1, ks2, lin_i32):
    bits = _threefry_bits(k0, k1, ks2, lin_i32.astype(jnp.uint32))
    fb = jax.lax.shift_right_logical(bits, jnp.uint32(9)) | jnp.uint32(
        0x3F800000)
    u01 = pltpu.bitcast(fb, jnp.float32) - jnp.float32(1.0)
    u = jnp.maximum(jnp.float32(_LO), u01 * jnp.float32(_SPAN)
                    + jnp.float32(_LO))
    return jnp.float32(_SQRT2) * jax.lax.erf_inv(u)


def _vad_kernel(idx_ref, kd_ref, tab_hbm, idxv_ref,
                mu_ref, lv_ref, std_ref, lat_ref, aug_ref,
                chunks, sem):
    tb = mu_ref.shape[0]
    dim = mu_ref.shape[1]
    nrows = tab_hbm.shape[0]
    row_elems = (_P + 1) * dim
    base_row = pl.program_id(0) * tb

    for i in range(tb):
        r = jnp.clip(idx_ref[base_row + i], 0, nrows - 1)
        c = pl.multiple_of((r >> 3) << 3, 8)
        pltpu.make_async_copy(tab_hbm.at[pl.ds(c, 8), :],
                              chunks.at[pl.ds(i * 8, 8), :], sem).start()

    k0 = kd_ref[0].astype(jnp.uint32)
    k1 = kd_ref[1].astype(jnp.uint32)
    ks2 = k0 ^ k1 ^ jnp.uint32(0x1BD11BDA)

    lin = (jax.lax.broadcasted_iota(jnp.int32, (tb, dim), 0) * row_elems
           + jax.lax.broadcasted_iota(jnp.int32, (tb, dim), 1)
           + (base_row * row_elems + _P * dim))
    eps_lat = _eps_from_counts(k0, k1, ks2, lin)

    pltpu.make_async_copy(tab_hbm.at[pl.ds(0, 8 * tb), :],
                          chunks.at[pl.ds(0, 8 * tb), :], sem).wait()
    blk = chunks[...].reshape(tb, 8, 2 * dim)
    rem = (idxv_ref[...] & 7).reshape(tb, 1, 1)
    sub = jax.lax.broadcasted_iota(jnp.int32, (tb, 8, 2 * dim), 1)
    picked = jnp.sum(blk * (sub == rem).astype(jnp.float32), axis=1)

    mu = picked[:, :dim]
    logvar = picked[:, dim:]
    std = jnp.exp(0.5 * logvar)
    mu_ref[...] = mu
    lv_ref[...] = logvar
    std_ref[...] = std
    lat_ref[...] = mu + eps_lat * std

    def aug_body(i, carry):
        b0 = pl.multiple_of(i * _CB, _CB)
        lin = (jax.lax.broadcasted_iota(jnp.int32, (_CB, _P, dim), 0)
               * row_elems
               + jax.lax.broadcasted_iota(jnp.int32, (_CB, _P, dim), 1) * dim
               + jax.lax.broadcasted_iota(jnp.int32, (_CB, _P, dim), 2)
               + (base_row + b0) * row_elems)
        eps = _eps_from_counts(k0, k1, ks2, lin)
        mu8 = mu_ref[pl.ds(b0, _CB), :]
        std8 = std_ref[pl.ds(b0, _CB), :]
        aug_ref[pl.ds(b0, _CB), :, :] = (mu8[:, None, :]
                                         + eps * std8[:, None, :])
        return carry

    jax.lax.fori_loop(0, tb // _CB, aug_body, 0)


def kernel(idx, tab_fused, eps_seed):
    b = int(idx.shape[0])
    n_pad, two_dim = tab_fused.shape
    dim = two_dim // 2

    kd = jax.lax.bitcast_convert_type(
        jax.random.key_data(jax.random.key(eps_seed)), jnp.int32)

    idx32 = idx.astype(jnp.int32)
    idx_col = idx32.reshape(b, 1)

    nsteps = 2 if b % 2 == 0 else 1
    tb = b // nsteps

    grid_spec = pltpu.PrefetchScalarGridSpec(
        num_scalar_prefetch=2,
        grid=(nsteps,),
        in_specs=[
            pl.BlockSpec(memory_space=pl.ANY),
            pl.BlockSpec((tb, 1), lambda g, *_: (g, 0)),
        ],
        out_specs=[
            pl.BlockSpec((tb, dim), lambda g, *_: (g, 0)),
            pl.BlockSpec((tb, dim), lambda g, *_: (g, 0)),
            pl.BlockSpec((tb, dim), lambda g, *_: (g, 0)),
            pl.BlockSpec((tb, dim), lambda g, *_: (g, 0)),
            pl.BlockSpec((tb, _P, dim), lambda g, *_: (g, 0, 0)),
        ],
        scratch_shapes=[
            pltpu.VMEM((tb * 8, two_dim), jnp.float32),
            pltpu.SemaphoreType.DMA,
        ],
    )
    out_shape = (tuple(jax.ShapeDtypeStruct((b, dim), jnp.float32)
                       for _ in range(4))
                 + (jax.ShapeDtypeStruct((b, _P, dim), jnp.float32),))
    mu, logvar, std, latent, latent_aug = pl.pallas_call(
        _vad_kernel,
        grid_spec=grid_spec,
        out_shape=out_shape,
        compiler_params=pltpu.CompilerParams(
            dimension_semantics=("parallel",)),
    )(idx32, kd, tab_fused, idx_col)

    return {'latent_code': latent,
            'latent_code_augment': latent_aug,
            'mu': mu, 'logvar': logvar, 'std': std}
